# Initial kernel scaffold; baseline (speedup 1.0000x reference)
import jax
import jax.numpy as jnp
from jax.experimental import pallas as pl


def kernel(x, edge_index, W1, b1, W_mu, b_mu, W_logstd, b_logstd):
    raise NotImplementedError("write your pallas kernel here")


# speedup vs baseline: 16.0255x; 16.0255x over previous
import functools

import jax
import jax.numpy as jnp
from jax import lax
from jax.experimental import pallas as pl
from jax.experimental.pallas import tpu as pltpu
from jax.experimental.pallas import tpu_sc as plsc

_NT = 16
_NC = 2
_NW = _NC * _NT
_CH = 128


def _pad_to(v, m):
    return ((v + m - 1) // m) * m


def _sc_degree(dst_blocks, zeros1, n_pad):
    k_chunks = dst_blocks.shape[1]
    rpt = n_pad // _NT
    mesh = plsc.VectorSubcoreMesh(core_axis_name="c", subcore_axis_name="s")

    @functools.partial(
        pl.kernel, mesh=mesh,
        out_type=jax.ShapeDtypeStruct((_NC, n_pad), jnp.float32),
        scratch_types=[
            pltpu.VMEM((k_chunks, _CH), jnp.int32),
            pltpu.VMEM((_CH,), jnp.float32),
            pltpu.VMEM_SHARED((n_pad,), jnp.float32),
        ],
    )
    def deg_kernel(dst_hbm, zero_hbm, out_hbm, dst_v, ones_v, acc_sh):
        c = lax.axis_index("c")
        s = lax.axis_index("s")
        wid = c * _NT + s
        r0 = s * rpt
        for i in range(_CH // 16):
            ones_v[pl.ds(i * 16, 16)] = jnp.ones((16,), jnp.float32)
        pltpu.sync_copy(zero_hbm.at[pl.ds(r0, rpt)], acc_sh.at[pl.ds(r0, rpt)])
        pltpu.sync_copy(dst_hbm.at[wid], dst_v)
        plsc.subcore_barrier()

        def body(j, carry):
            pltpu.sync_copy(ones_v, acc_sh.at[dst_v.at[j]], add=True)
            return carry

        lax.fori_loop(0, k_chunks, body, 0)
        plsc.subcore_barrier()
        pltpu.sync_copy(acc_sh.at[pl.ds(r0, rpt)], out_hbm.at[c, pl.ds(r0, rpt)])

    return deg_kernel(dst_blocks, zeros1)


def _sc_spmm(z_pad, src_blocks, dst_blocks, zeros2):
    n_pad, d = z_pad.shape
    k_chunks = src_blocks.shape[1]
    rpt = n_pad // _NT
    mesh = plsc.VectorSubcoreMesh(core_axis_name="c", subcore_axis_name="s")

    @functools.partial(
        pl.kernel, mesh=mesh,
        out_type=jax.ShapeDtypeStruct((_NC, n_pad, d), jnp.float32),
        scratch_types=[
            pltpu.VMEM((k_chunks, _CH), jnp.int32),
            pltpu.VMEM((k_chunks, _CH), jnp.int32),
            pltpu.VMEM((_CH, d), jnp.float32),
            pltpu.VMEM_SHARED((n_pad, d), jnp.float32),
            pltpu.SemaphoreType.DMA,
        ],
    )
    def spmm_kernel(z_hbm, src_hbm, dst_hbm, zero_hbm, out_hbm,
                    src_v, dst_v, rows_v, acc_sh, sem):
        c = lax.axis_index("c")
        s = lax.axis_index("s")
        wid = c * _NT + s
        r0 = s * rpt
        pltpu.sync_copy(zero_hbm.at[pl.ds(r0, rpt)], acc_sh.at[pl.ds(r0, rpt)])
        pltpu.sync_copy(src_hbm.at[wid], src_v)
        pltpu.sync_copy(dst_hbm.at[wid], dst_v)
        plsc.subcore_barrier()

        def body(j, carry):
            pltpu.async_copy(z_hbm.at[src_v.at[j]], rows_v, sem).wait()
            pltpu.sync_copy(rows_v, acc_sh.at[dst_v.at[j]], add=True)
            return carry

        lax.fori_loop(0, k_chunks, body, 0)
        plsc.subcore_barrier()
        pltpu.sync_copy(acc_sh.at[pl.ds(r0, rpt)],
                        out_hbm.at[c, pl.ds(r0, rpt)])

    return spmm_kernel(z_pad, src_blocks, dst_blocks, zeros2)


def _tc_layer1(x_pad, W1, deg0, deg1, block_n):
    n_pad, din = x_pad.shape
    dh = W1.shape[1]

    def body(x_ref, w_ref, d0_ref, d1_ref, z_ref, dis_ref):
        deg = d0_ref[...] + d1_ref[...] + 1.0
        dis = lax.rsqrt(deg)
        y = jnp.dot(x_ref[...], w_ref[...], preferred_element_type=jnp.float32)
        z_ref[...] = y * dis
        dis_ref[...] = dis

    return pl.pallas_call(
        body,
        grid=(n_pad // block_n,),
        in_specs=[
            pl.BlockSpec((block_n, din), lambda i: (i, 0)),
            pl.BlockSpec((din, dh), lambda i: (0, 0)),
            pl.BlockSpec((block_n, 1), lambda i: (i, 0)),
            pl.BlockSpec((block_n, 1), lambda i: (i, 0)),
        ],
        out_specs=[
            pl.BlockSpec((block_n, dh), lambda i: (i, 0)),
            pl.BlockSpec((block_n, 1), lambda i: (i, 0)),
        ],
        out_shape=[
            jax.ShapeDtypeStruct((n_pad, dh), jnp.float32),
            jax.ShapeDtypeStruct((n_pad, 1), jnp.float32),
        ],
    )(x_pad, W1, deg0, deg1)


def _tc_layer2(a0, a1, z1, dis, b1, Wcat, block_n):
    n_pad, dh = z1.shape
    dl = Wcat.shape[1]

    def body(a0_ref, a1_ref, z_ref, dis_ref, b_ref, w_ref, z2_ref):
        dis = dis_ref[...]
        h = (a0_ref[...] + a1_ref[...] + z_ref[...]) * dis + b_ref[...]
        h = jnp.maximum(h, 0.0)
        y2 = jnp.dot(h, w_ref[...], preferred_element_type=jnp.float32)
        z2_ref[...] = y2 * dis

    return pl.pallas_call(
        body,
        grid=(n_pad // block_n,),
        in_specs=[
            pl.BlockSpec((block_n, dh), lambda i: (i, 0)),
            pl.BlockSpec((block_n, dh), lambda i: (i, 0)),
            pl.BlockSpec((block_n, dh), lambda i: (i, 0)),
            pl.BlockSpec((block_n, 1), lambda i: (i, 0)),
            pl.BlockSpec((1, dh), lambda i: (0, 0)),
            pl.BlockSpec((dh, dl), lambda i: (0, 0)),
        ],
        out_specs=pl.BlockSpec((block_n, dl), lambda i: (i, 0)),
        out_shape=jax.ShapeDtypeStruct((n_pad, dl), jnp.float32),
    )(a0, a1, z1, dis, b1, Wcat)


def _tc_final(a0, a1, z2, dis, bcat, block_n):
    n_pad, dl = z2.shape

    def body(a0_ref, a1_ref, z_ref, dis_ref, b_ref, o_ref):
        o = (a0_ref[...] + a1_ref[...] + z_ref[...]) * dis_ref[...]
        o_ref[...] = o + b_ref[...]

    return pl.pallas_call(
        body,
        grid=(n_pad // block_n,),
        in_specs=[
            pl.BlockSpec((block_n, dl), lambda i: (i, 0)),
            pl.BlockSpec((block_n, dl), lambda i: (i, 0)),
            pl.BlockSpec((block_n, dl), lambda i: (i, 0)),
            pl.BlockSpec((block_n, 1), lambda i: (i, 0)),
            pl.BlockSpec((1, dl), lambda i: (0, 0)),
        ],
        out_specs=pl.BlockSpec((block_n, dl), lambda i: (i, 0)),
        out_shape=jax.ShapeDtypeStruct((n_pad, dl), jnp.float32),
    )(a0, a1, z2, dis, bcat)


def kernel(x, edge_index, W1, b1, W_mu, b_mu, W_logstd, b_logstd):
    n, din = x.shape
    e = edge_index.shape[1]
    dh = W1.shape[1]
    d_mu = W_mu.shape[1]
    dl = 128

    block_n = 1024
    n_pad = _pad_to(n + 1, block_n)
    e_pad = _pad_to(e, _NW * _CH)
    k_chunks = e_pad // (_NW * _CH)

    src = edge_index[0].astype(jnp.int32)
    dst = edge_index[1].astype(jnp.int32)
    pad_idx = jnp.full((e_pad - e,), n, jnp.int32)
    src_b = jnp.concatenate([src, pad_idx]).reshape(_NW, k_chunks, _CH)
    dst_b = jnp.concatenate([dst, pad_idx]).reshape(_NW, k_chunks, _CH)

    x_pad = jnp.zeros((n_pad, din), jnp.float32).at[:n].set(x)
    zeros1 = jnp.zeros((n_pad,), jnp.float32)
    zeros_h = jnp.zeros((n_pad, dh), jnp.float32)
    zeros_l = jnp.zeros((n_pad, dl), jnp.float32)

    Wcat = jnp.zeros((dh, dl), jnp.float32).at[:, : 2 * d_mu].set(
        jnp.concatenate([W_mu, W_logstd], axis=1))
    bcat = jnp.zeros((1, dl), jnp.float32).at[0, : 2 * d_mu].set(
        jnp.concatenate([b_mu, b_logstd]))
    b1r = b1.reshape(1, dh)

    deg = _sc_degree(dst_b, zeros1, n_pad)
    deg3 = deg.reshape(_NC, n_pad, 1)
    z1, dis = _tc_layer1(x_pad, W1, deg3[0], deg3[1], block_n)
    a1 = _sc_spmm(z1, src_b, dst_b, zeros_h)
    z2 = _tc_layer2(a1[0], a1[1], z1, dis, b1r, Wcat, block_n)
    a2 = _sc_spmm(z2, src_b, dst_b, zeros_l)
    out = _tc_final(a2[0], a2[1], z2, dis, bcat, block_n)

    mu = out[:n, :d_mu]
    logstd = out[:n, d_mu: 2 * d_mu]
    return (mu, logstd)
